# baseline (speedup 1.0000x reference)
import functools

import jax
import jax.numpy as jnp
from jax import lax
from jax.experimental import pallas as pl
from jax.experimental.pallas import tpu as pltpu
from jax.experimental.pallas import tpu_sc as plsc

N = 10000
E = 160000
HID = 512
OUTD = 256
G = 64

NSUB = 16
CH = 80
NIT = 128
EPT = NIT * CH
EPAD = NSUB * EPT
EPC = NSUB * NIT
NPAD = 10240
TRASH = N + 100
RPT = NPAD // NSUB
ZR = 128


def _make_sc_agg(nch, chunk_major):
    p_per_sc = nch // 2
    mesh = plsc.VectorSubcoreMesh(core_axis_name="c", subcore_axis_name="s")

    @functools.partial(
        pl.kernel,
        out_type=jax.ShapeDtypeStruct((nch * NPAD, 128), jnp.float32),
        mesh=mesh,
        scratch_types=[
            pltpu.VMEM((4, 2, CH), jnp.int32),
            pltpu.VMEM((2, CH, 128), jnp.float32),
            pltpu.VMEM((ZR, 128), jnp.float32),
            pltpu.VMEM_SHARED((NPAD, 128), jnp.float32),
        ] + [pltpu.SemaphoreType.DMA] * 6,
    )
    def sc_agg(h4, combo_hbm, out_hbm, idx_v, rows, zbuf, acc, *sems):
        gs, sts = sems[:2], sems[2:]
        c = lax.axis_index("c")
        s = lax.axis_index("s")

        def zrow(i, carry):
            for j in range(8):
                zbuf[i, pl.ds(j * 16, 16)] = jnp.zeros((16,), jnp.float32)
            return carry

        lax.fori_loop(0, ZR, zrow, 0)

        for p in range(p_per_sc):
            f = c * p_per_sc + p
            for j in range(RPT // ZR):
                pltpu.sync_copy(zbuf, acc.at[pl.ds(s * RPT + j * ZR, ZR)])
            pltpu.sync_copy(combo_hbm.at[f * EPC + s * NIT], idx_v.at[0])
            pltpu.async_copy(h4.at[idx_v.at[0, 0]], rows.at[0], gs[0])
            for u in range(1, 4):
                pltpu.async_copy(combo_hbm.at[f * EPC + s * NIT + u],
                                 idx_v.at[u], sts[u])
            plsc.subcore_barrier()

            def quad(q, carry):
                for u in range(4):
                    i = 4 * q + u
                    r = u % 2
                    nx = (u + 1) % 4
                    pltpu.make_async_copy(h4.at[idx_v.at[u, 0]],
                                          rows.at[r], gs[r]).wait()

                    def issue_next():
                        pltpu.make_async_copy(
                            combo_hbm.at[0], idx_v.at[nx], sts[nx]).wait()
                        pltpu.async_copy(h4.at[idx_v.at[nx, 0]],
                                         rows.at[r ^ 1], gs[r ^ 1])

                    if u < 3:
                        issue_next()
                    else:
                        @pl.when(q + 1 < NIT // 4)
                        def _():
                            issue_next()
                    pltpu.sync_copy(rows.at[r], acc.at[idx_v.at[u, 1]],
                                    add=True)

                    @pl.when(q + 1 < NIT // 4)
                    def _():
                        pltpu.async_copy(
                            combo_hbm.at[f * EPC + s * NIT + i + 4],
                            idx_v.at[u], sts[u])
                return carry

            lax.fori_loop(0, NIT // 4, quad, 0)
            plsc.subcore_barrier()
            pltpu.sync_copy(acc.at[pl.ds(s * RPT, RPT)],
                            out_hbm.at[pl.ds(f * NPAD + s * RPT, RPT)])

    return sc_agg


_SC_AGG = {}


def _get_sc_agg(nch, chunk_major):
    key = (nch, chunk_major)
    if key not in _SC_AGG:
        _SC_AGG[key] = _make_sc_agg(nch, chunk_major)
    return _SC_AGG[key]


_ROWS = 1000


def _k1_body(nch, x_chunked, h_ref, agg_ref, w1_ref, b1_ref, w2_ref, b2_ref,
             out_ref, st_ref):
    g = pl.program_id(0)
    if x_chunked:
        xv = jnp.concatenate([h_ref[i] for i in range(nch)], axis=1)
    else:
        xv = h_ref[...]
    xin = xv + jnp.concatenate([agg_ref[i] for i in range(nch)], axis=1)
    t = jnp.maximum(
        jnp.dot(xin, w1_ref[...], preferred_element_type=jnp.float32)
        + b1_ref[...], 0.0)
    h2 = jnp.dot(t, w2_ref[...], preferred_element_type=jnp.float32) \
        + b2_ref[...]
    out_ref[...] = h2
    s0 = jnp.sum(h2, axis=0)[None, :]
    s1 = jnp.sum(h2 * h2, axis=0)[None, :]
    blk = jnp.concatenate([s0, s1, jnp.zeros((6, HID), jnp.float32)], axis=0)

    @pl.when(g == 0)
    def _():
        st_ref[...] = blk

    @pl.when(g != 0)
    def _():
        st_ref[...] = st_ref[...] + blk


def _run_k1(h, agg, w1, b1, w2, b2, nch, din, x_chunked=False):
    grid = N // _ROWS
    if x_chunked:
        h_spec = pl.BlockSpec((nch, _ROWS, 128), lambda g: (0, g, 0))
    else:
        h_spec = pl.BlockSpec((_ROWS, din), lambda g: (g, 0))
    return pl.pallas_call(
        functools.partial(_k1_body, nch, x_chunked),
        grid=(grid,),
        in_specs=[
            h_spec,
            pl.BlockSpec((nch, _ROWS, 128), lambda g: (0, g, 0)),
            pl.BlockSpec((din, HID), lambda g: (0, 0)),
            pl.BlockSpec((1, HID), lambda g: (0, 0)),
            pl.BlockSpec((HID, HID), lambda g: (0, 0)),
            pl.BlockSpec((1, HID), lambda g: (0, 0)),
        ],
        out_specs=[
            pl.BlockSpec((_ROWS, HID), lambda g: (g, 0)),
            pl.BlockSpec((8, HID), lambda g: (0, 0)),
        ],
        out_shape=[
            jax.ShapeDtypeStruct((N, HID), jnp.float32),
            jax.ShapeDtypeStruct((8, HID), jnp.float32),
        ],
    )(h, agg, w1, b1.reshape(1, HID), w2, b2.reshape(1, HID))


def _k2_body(h2_ref, st_ref, gam_ref, bet_ref, out_ref):
    mean = st_ref[0:1, :] / N
    var = st_ref[1:2, :] / N - mean * mean
    rstd = lax.rsqrt(var + 1e-5)
    y = jnp.maximum(
        (h2_ref[...] - mean) * rstd * gam_ref[...] + bet_ref[...], 0.0)
    for i in range(HID // 128):
        out_ref[i] = y[:, 128 * i:128 * (i + 1)]


def _run_k2(h2, st, gamma, beta):
    grid = N // _ROWS
    return pl.pallas_call(
        _k2_body,
        grid=(grid,),
        in_specs=[
            pl.BlockSpec((_ROWS, HID), lambda g: (g, 0)),
            pl.BlockSpec((8, HID), lambda g: (0, 0)),
            pl.BlockSpec((1, HID), lambda g: (0, 0)),
            pl.BlockSpec((1, HID), lambda g: (0, 0)),
        ],
        out_specs=pl.BlockSpec((HID // 128, _ROWS, 128), lambda g: (0, g, 0)),
        out_shape=jax.ShapeDtypeStruct((HID // 128, N, 128), jnp.float32),
    )(h2, st, gamma.reshape(1, HID), beta.reshape(1, HID))


def _kpool_body(h2_ref, st_ref, gam_ref, bet_ref, bt_ref, wo_ref, bo_ref,
                out_ref, pool_ref, cnt_ref):
    g = pl.program_id(0)
    mean = st_ref[0:1, :] / N
    var = st_ref[1:2, :] / N - mean * mean
    rstd = lax.rsqrt(var + 1e-5)
    hn = jnp.maximum(
        (h2_ref[...] - mean) * rstd * gam_ref[...] + bet_ref[...], 0.0)
    ohT = (lax.broadcasted_iota(jnp.int32, (G, _ROWS), 0).astype(jnp.float32)
           == bt_ref[0]).astype(jnp.float32)
    ps = jnp.dot(ohT, hn, preferred_element_type=jnp.float32)
    cs = jnp.dot(ohT, jnp.ones((_ROWS, 128), jnp.float32),
                 preferred_element_type=jnp.float32)

    @pl.when(g == 0)
    def _():
        pool_ref[...] = ps
        cnt_ref[...] = cs

    @pl.when(g != 0)
    def _():
        pool_ref[...] = pool_ref[...] + ps
        cnt_ref[...] = cnt_ref[...] + cs

    pooled = pool_ref[...] / jnp.maximum(cnt_ref[:, 0:1], 1.0)
    out_ref[...] = jnp.dot(pooled, wo_ref[...],
                           preferred_element_type=jnp.float32) + bo_ref[...]


def _run_kpool(h2, st, gamma, beta, bt, wout, bout):
    grid = N // _ROWS
    return pl.pallas_call(
        _kpool_body,
        grid=(grid,),
        in_specs=[
            pl.BlockSpec((_ROWS, HID), lambda g: (g, 0)),
            pl.BlockSpec((8, HID), lambda g: (0, 0)),
            pl.BlockSpec((1, HID), lambda g: (0, 0)),
            pl.BlockSpec((1, HID), lambda g: (0, 0)),
            pl.BlockSpec((1, 1, _ROWS), lambda g: (g, 0, 0)),
            pl.BlockSpec((HID, OUTD), lambda g: (0, 0)),
            pl.BlockSpec((1, OUTD), lambda g: (0, 0)),
        ],
        out_specs=pl.BlockSpec((G, OUTD), lambda g: (0, 0)),
        out_shape=jax.ShapeDtypeStruct((G, OUTD), jnp.float32),
        scratch_shapes=[
            pltpu.VMEM((G, HID), jnp.float32),
            pltpu.VMEM((G, 128), jnp.float32),
        ],
    )(h2, st, gamma.reshape(1, HID), beta.reshape(1, HID), bt, wout,
      bout.reshape(1, OUTD))


def kernel(x, edge_index, batch,
           W1_0, b1_0, W2_0, b2_0, gamma_0, beta_0,
           W1_1, b1_1, W2_1, b2_1, gamma_1, beta_1,
           W1_2, b1_2, W2_2, b2_2, gamma_2, beta_2,
           Wout, bout):
    src = edge_index[0].astype(jnp.int32)
    dst = edge_index[1].astype(jnp.int32)
    bt = batch.astype(jnp.float32).reshape(N // _ROWS, 1, _ROWS)
    src_pad = jnp.concatenate([src, jnp.zeros((EPAD - E,), jnp.int32)])
    dst_pad = jnp.concatenate([dst, jnp.full((EPAD - E,), TRASH, jnp.int32)])
    idx2 = (src_pad[None, :] * 2
            + jnp.arange(2, dtype=jnp.int32)[:, None]).reshape(2, EPC, CH)
    idx4 = (src_pad[None, :]
            + (jnp.arange(4, dtype=jnp.int32) * N)[:, None]).reshape(4, EPC,
                                                                     CH)
    dst_c = jnp.broadcast_to(dst_pad.reshape(1, EPC, CH), (2, EPC, CH))
    combo2 = jnp.stack([idx2, dst_c], axis=2).reshape(2 * EPC, 2, CH)
    dst_c4 = jnp.broadcast_to(dst_pad.reshape(1, EPC, CH), (4, EPC, CH))
    combo4 = jnp.stack([idx4, dst_c4], axis=2).reshape(4 * EPC, 2, CH)

    agg = _get_sc_agg(2, False)(x.reshape(2 * N, 128),
                                combo2).reshape(2, NPAD, 128)
    h2, st = _run_k1(x, agg, W1_0, b1_0, W2_0, b2_0, nch=2, din=256)
    hn = _run_k2(h2, st, gamma_0, beta_0)

    agg = _get_sc_agg(4, True)(hn.reshape(4 * N, 128),
                               combo4).reshape(4, NPAD, 128)
    h2, st = _run_k1(hn, agg, W1_1, b1_1, W2_1, b2_1, nch=4, din=HID,
                     x_chunked=True)
    hn = _run_k2(h2, st, gamma_1, beta_1)

    agg = _get_sc_agg(4, True)(hn.reshape(4 * N, 128),
                               combo4).reshape(4, NPAD, 128)
    h2, st = _run_k1(hn, agg, W1_2, b1_2, W2_2, b2_2, nch=4, din=HID,
                     x_chunked=True)
    return _run_kpool(h2, st, gamma_2, beta_2, bt, Wout, bout)


# speedup vs baseline: 1.5243x; 1.5243x over previous
import functools

import jax
import jax.numpy as jnp
from jax import lax
from jax.experimental import pallas as pl
from jax.experimental.pallas import tpu as pltpu
from jax.experimental.pallas import tpu_sc as plsc

N = 10000
E = 160000
HID = 512
OUTD = 256
G = 64

NSUB = 16
CH = 80
NIT = 126
EPT = NIT * CH
EPAD = NSUB * EPT
EPC = NSUB * NIT
NPAD = 10240
TRASH = N + 100
RPT = NPAD // NSUB
ZR = 128


def _make_sc_agg(nch, chunk_major):
    p_per_sc = nch // 2
    mesh = plsc.VectorSubcoreMesh(core_axis_name="c", subcore_axis_name="s")

    @functools.partial(
        pl.kernel,
        out_type=jax.ShapeDtypeStruct((nch * NPAD, 128), jnp.float32),
        mesh=mesh,
        scratch_types=[
            pltpu.VMEM((2, 2, CH), jnp.int32),
            pltpu.VMEM((2, CH, 128), jnp.float32),
            pltpu.VMEM((ZR, 128), jnp.float32),
            pltpu.VMEM_SHARED((NPAD, 128), jnp.float32),
            pltpu.SemaphoreType.DMA,
            pltpu.SemaphoreType.DMA,
        ],
    )
    def sc_agg(h4, combo_hbm, out_hbm, idx_v, rows, zbuf, acc, *gs):
        c = lax.axis_index("c")
        s = lax.axis_index("s")

        def zrow(i, carry):
            for j in range(8):
                zbuf[i, pl.ds(j * 16, 16)] = jnp.zeros((16,), jnp.float32)
            return carry

        lax.fori_loop(0, ZR, zrow, 0)

        def stage(i, slot, f):
            pltpu.sync_copy(combo_hbm.at[f * EPC + s * NIT + i],
                            idx_v.at[slot])

        for p in range(p_per_sc):
            f = c * p_per_sc + p
            for j in range(RPT // ZR):
                pltpu.sync_copy(zbuf, acc.at[pl.ds(s * RPT + j * ZR, ZR)])
            stage(0, 0, f)
            pltpu.async_copy(h4.at[idx_v.at[0, 0]], rows.at[0], gs[0])
            stage(1, 1, f)
            plsc.subcore_barrier()

            def pair(gp, carry):
                for par in range(2):
                    i = 2 * gp + par
                    o = par ^ 1
                    pltpu.make_async_copy(h4.at[idx_v.at[par, 0]],
                                          rows.at[par], gs[par]).wait()

                    @pl.when(i + 1 < NIT)
                    def _():
                        pltpu.async_copy(h4.at[idx_v.at[o, 0]], rows.at[o],
                                         gs[o])
                    pltpu.sync_copy(rows.at[par], acc.at[idx_v.at[par, 1]],
                                    add=True)

                    @pl.when(i + 2 < NIT)
                    def _():
                        stage(i + 2, par, f)
                return carry

            lax.fori_loop(0, NIT // 2, pair, 0)
            plsc.subcore_barrier()
            pltpu.sync_copy(acc.at[pl.ds(s * RPT, RPT)],
                            out_hbm.at[pl.ds(f * NPAD + s * RPT, RPT)])

    return sc_agg


_SC_AGG = {}


def _get_sc_agg(nch, chunk_major):
    key = (nch, chunk_major)
    if key not in _SC_AGG:
        _SC_AGG[key] = _make_sc_agg(nch, chunk_major)
    return _SC_AGG[key]


_ROWS = 1000


def _k1_body(nch, x_chunked, h_ref, agg_ref, w1_ref, b1_ref, w2_ref, b2_ref,
             out_ref, st_ref):
    g = pl.program_id(0)
    if x_chunked:
        xv = jnp.concatenate([h_ref[i] for i in range(nch)], axis=1)
    else:
        xv = h_ref[...]
    xin = xv + jnp.concatenate([agg_ref[i] for i in range(nch)], axis=1)
    t = jnp.maximum(
        jnp.dot(xin, w1_ref[...], preferred_element_type=jnp.float32)
        + b1_ref[...], 0.0)
    h2 = jnp.dot(t, w2_ref[...], preferred_element_type=jnp.float32) \
        + b2_ref[...]
    out_ref[...] = h2
    s0 = jnp.sum(h2, axis=0)[None, :]
    s1 = jnp.sum(h2 * h2, axis=0)[None, :]
    blk = jnp.concatenate([s0, s1, jnp.zeros((6, HID), jnp.float32)], axis=0)

    @pl.when(g == 0)
    def _():
        st_ref[...] = blk

    @pl.when(g != 0)
    def _():
        st_ref[...] = st_ref[...] + blk


def _run_k1(h, agg, w1, b1, w2, b2, nch, din, x_chunked=False):
    grid = N // _ROWS
    if x_chunked:
        h_spec = pl.BlockSpec((nch, _ROWS, 128), lambda g: (0, g, 0))
    else:
        h_spec = pl.BlockSpec((_ROWS, din), lambda g: (g, 0))
    return pl.pallas_call(
        functools.partial(_k1_body, nch, x_chunked),
        grid=(grid,),
        in_specs=[
            h_spec,
            pl.BlockSpec((nch, _ROWS, 128), lambda g: (0, g, 0)),
            pl.BlockSpec((din, HID), lambda g: (0, 0)),
            pl.BlockSpec((1, HID), lambda g: (0, 0)),
            pl.BlockSpec((HID, HID), lambda g: (0, 0)),
            pl.BlockSpec((1, HID), lambda g: (0, 0)),
        ],
        out_specs=[
            pl.BlockSpec((_ROWS, HID), lambda g: (g, 0)),
            pl.BlockSpec((8, HID), lambda g: (0, 0)),
        ],
        out_shape=[
            jax.ShapeDtypeStruct((N, HID), jnp.float32),
            jax.ShapeDtypeStruct((8, HID), jnp.float32),
        ],
    )(h, agg, w1, b1.reshape(1, HID), w2, b2.reshape(1, HID))


def _k2_body(h2_ref, st_ref, gam_ref, bet_ref, out_ref):
    mean = st_ref[0:1, :] / N
    var = st_ref[1:2, :] / N - mean * mean
    rstd = lax.rsqrt(var + 1e-5)
    y = jnp.maximum(
        (h2_ref[...] - mean) * rstd * gam_ref[...] + bet_ref[...], 0.0)
    for i in range(HID // 128):
        out_ref[i] = y[:, 128 * i:128 * (i + 1)]


def _run_k2(h2, st, gamma, beta):
    grid = N // _ROWS
    return pl.pallas_call(
        _k2_body,
        grid=(grid,),
        in_specs=[
            pl.BlockSpec((_ROWS, HID), lambda g: (g, 0)),
            pl.BlockSpec((8, HID), lambda g: (0, 0)),
            pl.BlockSpec((1, HID), lambda g: (0, 0)),
            pl.BlockSpec((1, HID), lambda g: (0, 0)),
        ],
        out_specs=pl.BlockSpec((HID // 128, _ROWS, 128), lambda g: (0, g, 0)),
        out_shape=jax.ShapeDtypeStruct((HID // 128, N, 128), jnp.float32),
    )(h2, st, gamma.reshape(1, HID), beta.reshape(1, HID))


def _kpool_body(h2_ref, st_ref, gam_ref, bet_ref, bt_ref, wo_ref, bo_ref,
                out_ref, pool_ref, cnt_ref):
    g = pl.program_id(0)
    mean = st_ref[0:1, :] / N
    var = st_ref[1:2, :] / N - mean * mean
    rstd = lax.rsqrt(var + 1e-5)
    hn = jnp.maximum(
        (h2_ref[...] - mean) * rstd * gam_ref[...] + bet_ref[...], 0.0)
    ohT = (lax.broadcasted_iota(jnp.int32, (G, _ROWS), 0).astype(jnp.float32)
           == bt_ref[0]).astype(jnp.float32)
    ps = jnp.dot(ohT, hn, preferred_element_type=jnp.float32)
    cs = jnp.dot(ohT, jnp.ones((_ROWS, 128), jnp.float32),
                 preferred_element_type=jnp.float32)

    @pl.when(g == 0)
    def _():
        pool_ref[...] = ps
        cnt_ref[...] = cs

    @pl.when(g != 0)
    def _():
        pool_ref[...] = pool_ref[...] + ps
        cnt_ref[...] = cnt_ref[...] + cs

    pooled = pool_ref[...] / jnp.maximum(cnt_ref[:, 0:1], 1.0)
    out_ref[...] = jnp.dot(pooled, wo_ref[...],
                           preferred_element_type=jnp.float32) + bo_ref[...]


def _run_kpool(h2, st, gamma, beta, bt, wout, bout):
    grid = N // _ROWS
    return pl.pallas_call(
        _kpool_body,
        grid=(grid,),
        in_specs=[
            pl.BlockSpec((_ROWS, HID), lambda g: (g, 0)),
            pl.BlockSpec((8, HID), lambda g: (0, 0)),
            pl.BlockSpec((1, HID), lambda g: (0, 0)),
            pl.BlockSpec((1, HID), lambda g: (0, 0)),
            pl.BlockSpec((1, 1, _ROWS), lambda g: (g, 0, 0)),
            pl.BlockSpec((HID, OUTD), lambda g: (0, 0)),
            pl.BlockSpec((1, OUTD), lambda g: (0, 0)),
        ],
        out_specs=pl.BlockSpec((G, OUTD), lambda g: (0, 0)),
        out_shape=jax.ShapeDtypeStruct((G, OUTD), jnp.float32),
        scratch_shapes=[
            pltpu.VMEM((G, HID), jnp.float32),
            pltpu.VMEM((G, 128), jnp.float32),
        ],
    )(h2, st, gamma.reshape(1, HID), beta.reshape(1, HID), bt, wout,
      bout.reshape(1, OUTD))


def kernel(x, edge_index, batch,
           W1_0, b1_0, W2_0, b2_0, gamma_0, beta_0,
           W1_1, b1_1, W2_1, b2_1, gamma_1, beta_1,
           W1_2, b1_2, W2_2, b2_2, gamma_2, beta_2,
           Wout, bout):
    src = edge_index[0].astype(jnp.int32)
    dst = edge_index[1].astype(jnp.int32)
    bt = batch.astype(jnp.float32).reshape(N // _ROWS, 1, _ROWS)
    src_pad = jnp.concatenate([src, jnp.zeros((EPAD - E,), jnp.int32)])
    dst_pad = jnp.concatenate([dst, jnp.full((EPAD - E,), TRASH, jnp.int32)])
    idx2 = (src_pad[None, :] * 2
            + jnp.arange(2, dtype=jnp.int32)[:, None]).reshape(2, EPC, CH)
    idx4 = (src_pad[None, :]
            + (jnp.arange(4, dtype=jnp.int32) * N)[:, None]).reshape(4, EPC,
                                                                     CH)
    dst_c = jnp.broadcast_to(dst_pad.reshape(1, EPC, CH), (2, EPC, CH))
    combo2 = jnp.stack([idx2, dst_c], axis=2).reshape(2 * EPC, 2, CH)
    dst_c4 = jnp.broadcast_to(dst_pad.reshape(1, EPC, CH), (4, EPC, CH))
    combo4 = jnp.stack([idx4, dst_c4], axis=2).reshape(4 * EPC, 2, CH)

    agg = _get_sc_agg(2, False)(x.reshape(2 * N, 128),
                                combo2).reshape(2, NPAD, 128)
    h2, st = _run_k1(x, agg, W1_0, b1_0, W2_0, b2_0, nch=2, din=256)
    hn = _run_k2(h2, st, gamma_0, beta_0)

    agg = _get_sc_agg(4, True)(hn.reshape(4 * N, 128),
                               combo4).reshape(4, NPAD, 128)
    h2, st = _run_k1(hn, agg, W1_1, b1_1, W2_1, b2_1, nch=4, din=HID,
                     x_chunked=True)
    hn = _run_k2(h2, st, gamma_1, beta_1)

    agg = _get_sc_agg(4, True)(hn.reshape(4 * N, 128),
                               combo4).reshape(4, NPAD, 128)
    h2, st = _run_k1(hn, agg, W1_2, b1_2, W2_2, b2_2, nch=4, din=HID,
                     x_chunked=True)
    return _run_kpool(h2, st, gamma_2, beta_2, bt, Wout, bout)


# speedup vs baseline: 1.5293x; 1.0033x over previous
import functools

import jax
import jax.numpy as jnp
from jax import lax
from jax.experimental import pallas as pl
from jax.experimental.pallas import tpu as pltpu
from jax.experimental.pallas import tpu_sc as plsc

N = 10000
E = 160000
HID = 512
OUTD = 256
G = 64

NSUB = 16
CH = 80
NIT = 126
EPT = NIT * CH
EPAD = NSUB * EPT
EPC = NSUB * NIT
NPAD = 10240
TRASH = N + 100
RPT = NPAD // NSUB
ZR = 128


def _make_sc_agg(nch, chunk_major):
    p_per_sc = nch // 2
    mesh = plsc.VectorSubcoreMesh(core_axis_name="c", subcore_axis_name="s")

    @functools.partial(
        pl.kernel,
        out_type=jax.ShapeDtypeStruct((nch * NPAD, 128), jnp.float32),
        mesh=mesh,
        scratch_types=[
            pltpu.VMEM((2, 2, CH), jnp.int32),
            pltpu.VMEM((2, CH, 128), jnp.float32),
            pltpu.VMEM((ZR, 128), jnp.float32),
            pltpu.VMEM_SHARED((NPAD, 128), jnp.float32),
            pltpu.SemaphoreType.DMA,
            pltpu.SemaphoreType.DMA,
        ],
    )
    def sc_agg(h4, combo_hbm, out_hbm, idx_v, rows, zbuf, acc, *gs):
        c = lax.axis_index("c")
        s = lax.axis_index("s")

        def zrow(i, carry):
            for j in range(8):
                zbuf[i, pl.ds(j * 16, 16)] = jnp.zeros((16,), jnp.float32)
            return carry

        lax.fori_loop(0, ZR, zrow, 0)

        def stage(i, slot, f):
            pltpu.sync_copy(combo_hbm.at[f * EPC + s * NIT + i],
                            idx_v.at[slot])

        for p in range(p_per_sc):
            f = c * p_per_sc + p
            for j in range(RPT // ZR):
                pltpu.sync_copy(zbuf, acc.at[pl.ds(s * RPT + j * ZR, ZR)])
            stage(0, 0, f)
            pltpu.async_copy(h4.at[idx_v.at[0, 0]], rows.at[0], gs[0])
            stage(1, 1, f)
            plsc.subcore_barrier()

            def pair(gp, carry):
                for par in range(2):
                    i = 2 * gp + par
                    o = par ^ 1
                    pltpu.make_async_copy(h4.at[idx_v.at[par, 0]],
                                          rows.at[par], gs[par]).wait()

                    @pl.when(i + 1 < NIT)
                    def _():
                        pltpu.async_copy(h4.at[idx_v.at[o, 0]], rows.at[o],
                                         gs[o])
                    pltpu.sync_copy(rows.at[par], acc.at[idx_v.at[par, 1]],
                                    add=True)

                    @pl.when(i + 2 < NIT)
                    def _():
                        stage(i + 2, par, f)
                return carry

            lax.fori_loop(0, NIT // 2, pair, 0)
            plsc.subcore_barrier()
            pltpu.sync_copy(acc.at[pl.ds(s * RPT, RPT)],
                            out_hbm.at[pl.ds(f * NPAD + s * RPT, RPT)])

    return sc_agg


_SC_AGG = {}


def _get_sc_agg(nch, chunk_major):
    key = (nch, chunk_major)
    if key not in _SC_AGG:
        _SC_AGG[key] = _make_sc_agg(nch, chunk_major)
    return _SC_AGG[key]


_ROWS = 2000


def _k1_body(nch, x_chunked, h_ref, agg_ref, w1_ref, b1_ref, w2_ref, b2_ref,
             out_ref, st_ref):
    g = pl.program_id(0)
    if x_chunked:
        xv = jnp.concatenate([h_ref[i] for i in range(nch)], axis=1)
    else:
        xv = h_ref[...]
    xin = xv + jnp.concatenate([agg_ref[i] for i in range(nch)], axis=1)
    t = jnp.maximum(
        jnp.dot(xin, w1_ref[...], preferred_element_type=jnp.float32)
        + b1_ref[...], 0.0)
    h2 = jnp.dot(t, w2_ref[...], preferred_element_type=jnp.float32) \
        + b2_ref[...]
    out_ref[...] = h2
    s0 = jnp.sum(h2, axis=0)[None, :]
    s1 = jnp.sum(h2 * h2, axis=0)[None, :]
    blk = jnp.concatenate([s0, s1, jnp.zeros((6, HID), jnp.float32)], axis=0)

    @pl.when(g == 0)
    def _():
        st_ref[...] = blk

    @pl.when(g != 0)
    def _():
        st_ref[...] = st_ref[...] + blk


def _run_k1(h, agg, w1, b1, w2, b2, nch, din, x_chunked=False):
    grid = N // _ROWS
    if x_chunked:
        h_spec = pl.BlockSpec((nch, _ROWS, 128), lambda g: (0, g, 0))
    else:
        h_spec = pl.BlockSpec((_ROWS, din), lambda g: (g, 0))
    return pl.pallas_call(
        functools.partial(_k1_body, nch, x_chunked),
        grid=(grid,),
        in_specs=[
            h_spec,
            pl.BlockSpec((nch, _ROWS, 128), lambda g: (0, g, 0)),
            pl.BlockSpec((din, HID), lambda g: (0, 0)),
            pl.BlockSpec((1, HID), lambda g: (0, 0)),
            pl.BlockSpec((HID, HID), lambda g: (0, 0)),
            pl.BlockSpec((1, HID), lambda g: (0, 0)),
        ],
        out_specs=[
            pl.BlockSpec((_ROWS, HID), lambda g: (g, 0)),
            pl.BlockSpec((8, HID), lambda g: (0, 0)),
        ],
        out_shape=[
            jax.ShapeDtypeStruct((N, HID), jnp.float32),
            jax.ShapeDtypeStruct((8, HID), jnp.float32),
        ],
    )(h, agg, w1, b1.reshape(1, HID), w2, b2.reshape(1, HID))


def _k2_body(h2_ref, st_ref, gam_ref, bet_ref, out_ref):
    mean = st_ref[0:1, :] / N
    var = st_ref[1:2, :] / N - mean * mean
    rstd = lax.rsqrt(var + 1e-5)
    y = jnp.maximum(
        (h2_ref[...] - mean) * rstd * gam_ref[...] + bet_ref[...], 0.0)
    for i in range(HID // 128):
        out_ref[i] = y[:, 128 * i:128 * (i + 1)]


def _run_k2(h2, st, gamma, beta):
    grid = N // _ROWS
    return pl.pallas_call(
        _k2_body,
        grid=(grid,),
        in_specs=[
            pl.BlockSpec((_ROWS, HID), lambda g: (g, 0)),
            pl.BlockSpec((8, HID), lambda g: (0, 0)),
            pl.BlockSpec((1, HID), lambda g: (0, 0)),
            pl.BlockSpec((1, HID), lambda g: (0, 0)),
        ],
        out_specs=pl.BlockSpec((HID // 128, _ROWS, 128), lambda g: (0, g, 0)),
        out_shape=jax.ShapeDtypeStruct((HID // 128, N, 128), jnp.float32),
    )(h2, st, gamma.reshape(1, HID), beta.reshape(1, HID))


def _kpool_body(h2_ref, st_ref, gam_ref, bet_ref, bt_ref, wo_ref, bo_ref,
                out_ref, pool_ref, cnt_ref):
    g = pl.program_id(0)
    mean = st_ref[0:1, :] / N
    var = st_ref[1:2, :] / N - mean * mean
    rstd = lax.rsqrt(var + 1e-5)
    hn = jnp.maximum(
        (h2_ref[...] - mean) * rstd * gam_ref[...] + bet_ref[...], 0.0)
    ohT = (lax.broadcasted_iota(jnp.int32, (G, _ROWS), 0).astype(jnp.float32)
           == bt_ref[0]).astype(jnp.float32)
    ps = jnp.dot(ohT, hn, preferred_element_type=jnp.float32)
    cs = jnp.dot(ohT, jnp.ones((_ROWS, 128), jnp.float32),
                 preferred_element_type=jnp.float32)

    @pl.when(g == 0)
    def _():
        pool_ref[...] = ps
        cnt_ref[...] = cs

    @pl.when(g != 0)
    def _():
        pool_ref[...] = pool_ref[...] + ps
        cnt_ref[...] = cnt_ref[...] + cs

    pooled = pool_ref[...] / jnp.maximum(cnt_ref[:, 0:1], 1.0)
    out_ref[...] = jnp.dot(pooled, wo_ref[...],
                           preferred_element_type=jnp.float32) + bo_ref[...]


def _run_kpool(h2, st, gamma, beta, bt, wout, bout):
    grid = N // _ROWS
    return pl.pallas_call(
        _kpool_body,
        grid=(grid,),
        in_specs=[
            pl.BlockSpec((_ROWS, HID), lambda g: (g, 0)),
            pl.BlockSpec((8, HID), lambda g: (0, 0)),
            pl.BlockSpec((1, HID), lambda g: (0, 0)),
            pl.BlockSpec((1, HID), lambda g: (0, 0)),
            pl.BlockSpec((1, 1, _ROWS), lambda g: (g, 0, 0)),
            pl.BlockSpec((HID, OUTD), lambda g: (0, 0)),
            pl.BlockSpec((1, OUTD), lambda g: (0, 0)),
        ],
        out_specs=pl.BlockSpec((G, OUTD), lambda g: (0, 0)),
        out_shape=jax.ShapeDtypeStruct((G, OUTD), jnp.float32),
        scratch_shapes=[
            pltpu.VMEM((G, HID), jnp.float32),
            pltpu.VMEM((G, 128), jnp.float32),
        ],
    )(h2, st, gamma.reshape(1, HID), beta.reshape(1, HID), bt, wout,
      bout.reshape(1, OUTD))


def kernel(x, edge_index, batch,
           W1_0, b1_0, W2_0, b2_0, gamma_0, beta_0,
           W1_1, b1_1, W2_1, b2_1, gamma_1, beta_1,
           W1_2, b1_2, W2_2, b2_2, gamma_2, beta_2,
           Wout, bout):
    src = edge_index[0].astype(jnp.int32)
    dst = edge_index[1].astype(jnp.int32)
    bt = batch.astype(jnp.float32).reshape(N // _ROWS, 1, _ROWS)
    src_pad = jnp.concatenate([src, jnp.zeros((EPAD - E,), jnp.int32)])
    dst_pad = jnp.concatenate([dst, jnp.full((EPAD - E,), TRASH, jnp.int32)])
    idx2 = (src_pad[None, :] * 2
            + jnp.arange(2, dtype=jnp.int32)[:, None]).reshape(2, EPC, CH)
    idx4 = (src_pad[None, :]
            + (jnp.arange(4, dtype=jnp.int32) * N)[:, None]).reshape(4, EPC,
                                                                     CH)
    dst_c = jnp.broadcast_to(dst_pad.reshape(1, EPC, CH), (2, EPC, CH))
    combo2 = jnp.stack([idx2, dst_c], axis=2).reshape(2 * EPC, 2, CH)
    dst_c4 = jnp.broadcast_to(dst_pad.reshape(1, EPC, CH), (4, EPC, CH))
    combo4 = jnp.stack([idx4, dst_c4], axis=2).reshape(4 * EPC, 2, CH)

    agg = _get_sc_agg(2, False)(x.reshape(2 * N, 128),
                                combo2).reshape(2, NPAD, 128)
    h2, st = _run_k1(x, agg, W1_0, b1_0, W2_0, b2_0, nch=2, din=256)
    hn = _run_k2(h2, st, gamma_0, beta_0)

    agg = _get_sc_agg(4, True)(hn.reshape(4 * N, 128),
                               combo4).reshape(4, NPAD, 128)
    h2, st = _run_k1(hn, agg, W1_1, b1_1, W2_1, b2_1, nch=4, din=HID,
                     x_chunked=True)
    hn = _run_k2(h2, st, gamma_1, beta_1)

    agg = _get_sc_agg(4, True)(hn.reshape(4 * N, 128),
                               combo4).reshape(4, NPAD, 128)
    h2, st = _run_k1(hn, agg, W1_2, b1_2, W2_2, b2_2, nch=4, din=HID,
                     x_chunked=True)
    return _run_kpool(h2, st, gamma_2, beta_2, bt, Wout, bout)
